# baseline (speedup 1.0000x reference)
import functools

import jax
import jax.numpy as jnp
from jax import lax
from jax.experimental import pallas as pl
from jax.experimental.pallas import tpu as pltpu
from jax.experimental.pallas import tpu_sc as plsc

_NC = 2
_NS = 16
_NW = _NC * _NS


def _sc_gather(x_flat, flat_idx, n_idx, d):
    per_w = n_idx // _NW
    chunk = 320
    n_chunks = per_w // chunk
    nbuf = 3

    mesh = plsc.VectorSubcoreMesh(core_axis_name="c", subcore_axis_name="s")

    @functools.partial(
        pl.kernel,
        mesh=mesh,
        out_type=jax.ShapeDtypeStruct((n_idx, d), x_flat.dtype),
        scratch_types=[
            pltpu.VMEM((per_w,), jnp.int32),
            pltpu.VMEM((nbuf, chunk, d), jnp.float32),
            pltpu.SemaphoreType.DMA,
            pltpu.SemaphoreType.DMA,
        ],
    )
    def kern(x_hbm, idx_hbm, out_hbm, idx_v, rows_v, gsem, osem):
        wid = lax.axis_index("s") * _NC + lax.axis_index("c")
        base = wid * per_w
        pltpu.sync_copy(idx_hbm.at[pl.ds(base, per_w)], idx_v)

        def mk_g(c):
            return pltpu.make_async_copy(
                x_hbm.at[idx_v.at[pl.ds(c * chunk, chunk)]],
                rows_v.at[c % nbuf], gsem)

        def mk_o(c):
            return pltpu.make_async_copy(
                rows_v.at[c % nbuf],
                out_hbm.at[pl.ds(base + c * chunk, chunk)], osem)

        for c in range(min(nbuf, n_chunks)):
            mk_g(c).start()
        o_waited = 0
        for c in range(n_chunks):
            mk_g(c).wait()
            mk_o(c).start()
            if c + nbuf < n_chunks:
                mk_o(c).wait()
                o_waited += 1
                mk_g(c + nbuf).start()
        for c in range(n_chunks - o_waited):
            mk_o(c).wait()

    return kern(x_flat, flat_idx)


def kernel(x, idxs):
    b, n, d = x.shape
    k = idxs.shape[1]
    n_idx = b * k
    x_flat = x.reshape(b * n, d)
    tidx = (
        idxs.astype(jnp.int32) + (jnp.arange(b, dtype=jnp.int32) * n)[:, None]
    ).T.reshape(n_idx)
    out = _sc_gather(x_flat, tidx, n_idx, d)
    return out.reshape(k, b, d).transpose(1, 0, 2)


# speedup vs baseline: 1.0115x; 1.0115x over previous
import functools

import jax
import jax.numpy as jnp
from jax import lax
from jax.experimental import pallas as pl
from jax.experimental.pallas import tpu as pltpu
from jax.experimental.pallas import tpu_sc as plsc

_NC = 2
_NS = 16
_NW = _NC * _NS


def _sc_gather(x_flat, flat_idx, n_idx, d):
    per_w = n_idx // _NW
    chunk = 400
    n_chunks = per_w // chunk
    nbuf = 2

    mesh = plsc.VectorSubcoreMesh(core_axis_name="c", subcore_axis_name="s")

    @functools.partial(
        pl.kernel,
        mesh=mesh,
        out_type=jax.ShapeDtypeStruct((n_idx, d), x_flat.dtype),
        scratch_types=[
            pltpu.VMEM((per_w,), jnp.int32),
            pltpu.VMEM((nbuf, chunk, d), jnp.float32),
            pltpu.SemaphoreType.DMA,
            pltpu.SemaphoreType.DMA,
        ],
    )
    def kern(x_hbm, idx_hbm, out_hbm, idx_v, rows_v, gsem, osem):
        wid = lax.axis_index("s") * _NC + lax.axis_index("c")
        base = wid * per_w
        pltpu.sync_copy(idx_hbm.at[pl.ds(base, per_w)], idx_v)

        def mk_g(c):
            return pltpu.make_async_copy(
                x_hbm.at[idx_v.at[pl.ds(c * chunk, chunk)]],
                rows_v.at[c % nbuf], gsem)

        def mk_o(c):
            return pltpu.make_async_copy(
                rows_v.at[c % nbuf],
                out_hbm.at[pl.ds(base + c * chunk, chunk)], osem)

        for c in range(min(nbuf, n_chunks)):
            mk_g(c).start()
        o_waited = 0
        for c in range(n_chunks):
            mk_g(c).wait()
            mk_o(c).start()
            if c + nbuf < n_chunks:
                mk_o(c).wait()
                o_waited += 1
                mk_g(c + nbuf).start()
        for c in range(n_chunks - o_waited):
            mk_o(c).wait()

    return kern(x_flat, flat_idx)


def kernel(x, idxs):
    b, n, d = x.shape
    k = idxs.shape[1]
    n_idx = b * k
    x_flat = x.reshape(b * n, d)
    tidx = (
        idxs.astype(jnp.int32) + (jnp.arange(b, dtype=jnp.int32) * n)[:, None]
    ).T.reshape(n_idx)
    out = _sc_gather(x_flat, tidx, n_idx, d)
    return out.reshape(k, b, d).transpose(1, 0, 2)
